# Initial kernel scaffold; baseline (speedup 1.0000x reference)
import jax
import jax.numpy as jnp
from jax.experimental import pallas as pl


def kernel(x, edge_index, W1, b1, W2, b2):
    raise NotImplementedError("write your pallas kernel here")


# speedup vs baseline: 19.8573x; 19.8573x over previous
import functools

import jax
import jax.numpy as jnp
from jax import lax
from jax.experimental import pallas as pl
from jax.experimental.pallas import tpu as pltpu
from jax.experimental.pallas import tpu_sc as plsc

_CHUNK = 128
_NW = 32


def _wid_and_bounds(nchunks):
    c = lax.axis_index("c")
    s = lax.axis_index("s")
    wid = c * 16 + s
    c0 = (wid * nchunks) // _NW
    c1 = ((wid + 1) * nchunks) // _NW
    return s, c, c0, c1


def _row_split(n):
    rpt = ((n + 15) // 16 + 7) // 8 * 8
    last = n - 15 * rpt
    return rpt, last


def _tile_copy(s, n, src_fn, dst_fn):
    rpt, last = _row_split(n)
    row0 = pl.multiple_of(s * rpt, 8)

    @pl.when(s < 15)
    def _():
        pltpu.sync_copy(src_fn(row0, rpt), dst_fn(row0, rpt))

    @pl.when(s == 15)
    def _():
        pltpu.sync_copy(src_fn(15 * rpt, last), dst_fn(15 * rpt, last))


@functools.lru_cache(maxsize=None)
def _make_degree(n, e):
    nchunks = e // _CHUNK
    mesh = plsc.VectorSubcoreMesh(core_axis_name="c", subcore_axis_name="s",
                                  num_cores=2, num_subcores=16)

    @functools.partial(
        pl.kernel,
        out_type=jax.ShapeDtypeStruct((2, n), jnp.float32),
        mesh=mesh,
        scratch_types=[
            pltpu.VMEM((_CHUNK,), jnp.int32),
            pltpu.VMEM((_CHUNK,), jnp.float32),
            pltpu.VMEM_SHARED((n,), jnp.float32),
        ],
        compiler_params=pltpu.CompilerParams(use_tc_tiling_on_sc=False),
    )
    def degree(dst_hbm, zeros_hbm, out_hbm, idx_v, ones_v, deg_sh):
        s, c, c0, c1 = _wid_and_bounds(nchunks)
        _tile_copy(s, n,
                   lambda r0, m: zeros_hbm.at[pl.ds(r0, m)],
                   lambda r0, m: deg_sh.at[pl.ds(r0, m)])
        for i in range(_CHUNK // 16):
            ones_v[pl.ds(i * 16, 16)] = jnp.ones((16,), jnp.float32)
        plsc.subcore_barrier()

        def body(ch, carry):
            off = pl.multiple_of(ch * _CHUNK, 8)
            pltpu.sync_copy(dst_hbm.at[pl.ds(off, _CHUNK)], idx_v)
            pltpu.sync_copy(ones_v, deg_sh.at[idx_v], add=True)
            return carry

        lax.fori_loop(c0, c1, body, 0)
        plsc.subcore_barrier()
        _tile_copy(s, n,
                   lambda r0, m: deg_sh.at[pl.ds(r0, m)],
                   lambda r0, m: out_hbm.at[c, pl.ds(r0, m)])

    return degree


@functools.lru_cache(maxsize=None)
def _make_agg(n, e, f):
    nchunks = e // _CHUNK
    mesh = plsc.VectorSubcoreMesh(core_axis_name="c", subcore_axis_name="s",
                                  num_cores=2, num_subcores=16)

    @functools.partial(
        pl.kernel,
        out_type=jax.ShapeDtypeStruct((2, n, f), jnp.float32),
        mesh=mesh,
        scratch_types=[
            pltpu.VMEM((_CHUNK,), jnp.int32),
            pltpu.VMEM((_CHUNK,), jnp.int32),
            pltpu.VMEM((_CHUNK, f), jnp.float32),
            pltpu.VMEM_SHARED((n, f), jnp.float32),
            pltpu.SemaphoreType.DMA,
        ],
        compiler_params=pltpu.CompilerParams(use_tc_tiling_on_sc=False),
    )
    def agg(hs_hbm, src_hbm, dst_hbm, out_hbm, sidx, didx, rows, s_sh, sem):
        s, c, c0, c1 = _wid_and_bounds(nchunks)
        _tile_copy(s, n,
                   lambda r0, m: hs_hbm.at[pl.ds(r0, m), :],
                   lambda r0, m: s_sh.at[pl.ds(r0, m), :])
        plsc.subcore_barrier()

        def body(ch, carry):
            off = pl.multiple_of(ch * _CHUNK, 8)
            pltpu.sync_copy(src_hbm.at[pl.ds(off, _CHUNK)], sidx)
            pltpu.sync_copy(dst_hbm.at[pl.ds(off, _CHUNK)], didx)
            pltpu.async_copy(hs_hbm.at[sidx], rows, sem).wait()
            pltpu.sync_copy(rows, s_sh.at[didx], add=True)
            return carry

        lax.fori_loop(c0, c1, body, 0)
        plsc.subcore_barrier()
        _tile_copy(s, n,
                   lambda r0, m: s_sh.at[pl.ds(r0, m), :],
                   lambda r0, m: out_hbm.at[c, pl.ds(r0, m), :])

    return agg


def _prep1_body(x_ref, w1_ref, degp_ref, hs_ref, dinv_ref):
    deg = degp_ref[0] + degp_ref[1] + 1.0
    dinv = lax.rsqrt(deg)
    h = jnp.dot(x_ref[...], w1_ref[...], preferred_element_type=jnp.float32)
    hs_ref[...] = h * dinv
    dinv_ref[...] = dinv


def _mid_body(s1_ref, hs1_ref, dinv_ref, b1_ref, w2_ref, hs2_ref):
    dinv = dinv_ref[...]
    agg = s1_ref[0] + s1_ref[1] - hs1_ref[...]
    h = jnp.maximum(agg * dinv + b1_ref[...], 0.0)
    hs2_ref[...] = jnp.dot(h, w2_ref[...],
                           preferred_element_type=jnp.float32) * dinv


def _final_body(s2_ref, hs2_ref, dinv_ref, b2_ref, out_ref):
    o = (s2_ref[0] + s2_ref[1] - hs2_ref[...]) * dinv_ref[...] + b2_ref[...]
    m = jnp.max(o, axis=1, keepdims=True)
    z = o - m
    out_ref[...] = z - jnp.log(jnp.sum(jnp.exp(z), axis=1, keepdims=True))


def kernel(x, edge_index, W1, b1, W2, b2):
    n, d = x.shape
    h_dim = W1.shape[1]
    c_dim = W2.shape[1]
    e = edge_index.shape[1]
    src = edge_index[0]
    dst = edge_index[1]
    f32 = jnp.float32

    zeros_n = jnp.zeros((n,), f32)
    degp = _make_degree(n, e)(dst, zeros_n)[:, :, None]

    bm = 1000
    grid = (n // bm,)
    hs1, dinv = pl.pallas_call(
        _prep1_body,
        grid=grid,
        in_specs=[
            pl.BlockSpec((bm, d), lambda i: (i, 0)),
            pl.BlockSpec((d, h_dim), lambda i: (0, 0)),
            pl.BlockSpec((2, bm, 1), lambda i: (0, i, 0)),
        ],
        out_specs=[
            pl.BlockSpec((bm, h_dim), lambda i: (i, 0)),
            pl.BlockSpec((bm, 1), lambda i: (i, 0)),
        ],
        out_shape=[
            jax.ShapeDtypeStruct((n, h_dim), f32),
            jax.ShapeDtypeStruct((n, 1), f32),
        ],
    )(x, W1, degp)

    s1 = _make_agg(n, e, h_dim)(hs1, src, dst)

    hs2 = pl.pallas_call(
        _mid_body,
        grid=grid,
        in_specs=[
            pl.BlockSpec((2, bm, h_dim), lambda i: (0, i, 0)),
            pl.BlockSpec((bm, h_dim), lambda i: (i, 0)),
            pl.BlockSpec((bm, 1), lambda i: (i, 0)),
            pl.BlockSpec((1, h_dim), lambda i: (0, 0)),
            pl.BlockSpec((h_dim, c_dim), lambda i: (0, 0)),
        ],
        out_specs=pl.BlockSpec((bm, c_dim), lambda i: (i, 0)),
        out_shape=jax.ShapeDtypeStruct((n, c_dim), f32),
    )(s1, hs1, dinv, b1.reshape(1, h_dim), W2)

    s2 = _make_agg(n, e, c_dim)(hs2, src, dst)

    out = pl.pallas_call(
        _final_body,
        grid=grid,
        in_specs=[
            pl.BlockSpec((2, bm, c_dim), lambda i: (0, i, 0)),
            pl.BlockSpec((bm, c_dim), lambda i: (i, 0)),
            pl.BlockSpec((bm, 1), lambda i: (i, 0)),
            pl.BlockSpec((1, c_dim), lambda i: (0, 0)),
        ],
        out_specs=pl.BlockSpec((bm, c_dim), lambda i: (i, 0)),
        out_shape=jax.ShapeDtypeStruct((n, c_dim), f32),
    )(s2, hs2, dinv, b2.reshape(1, c_dim))

    return out


# speedup vs baseline: 40.8393x; 2.0566x over previous
import functools

import jax
import jax.numpy as jnp
from jax import lax
from jax.experimental import pallas as pl
from jax.experimental.pallas import tpu as pltpu
from jax.experimental.pallas import tpu_sc as plsc

_CHUNK = 128
_NW = 32


def _worker_id():
    return lax.axis_index("c") * 16 + lax.axis_index("s")


def _row_split(n):
    rpt = ((n + 15) // 16 + 7) // 8 * 8
    last = n - 15 * rpt
    return rpt, last


def _tile_copy(s, n, src_fn, dst_fn):
    rpt, last = _row_split(n)
    row0 = pl.multiple_of(s * rpt, 8)

    @pl.when(s < 15)
    def _():
        pltpu.sync_copy(src_fn(row0, rpt), dst_fn(row0, rpt))

    @pl.when(s == 15)
    def _():
        pltpu.sync_copy(src_fn(15 * rpt, last), dst_fn(15 * rpt, last))


def _mesh():
    return plsc.VectorSubcoreMesh(core_axis_name="c", subcore_axis_name="s",
                                  num_cores=2, num_subcores=16)


@functools.lru_cache(maxsize=None)
def _make_degree(n, e):
    nchunks = e // _CHUNK
    base = nchunks // _NW
    extra = nchunks - base * _NW

    @functools.partial(
        pl.kernel,
        out_type=jax.ShapeDtypeStruct((2, n), jnp.float32),
        mesh=_mesh(),
        scratch_types=[
            pltpu.VMEM((base, _CHUNK), jnp.int32),
            pltpu.VMEM((_CHUNK,), jnp.int32),
            pltpu.VMEM((_CHUNK,), jnp.float32),
            pltpu.VMEM_SHARED((n,), jnp.float32),
        ],
        compiler_params=pltpu.CompilerParams(use_tc_tiling_on_sc=False),
    )
    def degree(dst2_hbm, zeros_hbm, out_hbm, didx, xidx, ones_v, deg_sh):
        c = lax.axis_index("c")
        s = lax.axis_index("s")
        wid = _worker_id()
        c0 = wid * base
        pltpu.sync_copy(dst2_hbm.at[pl.ds(c0, base), :], didx)
        _tile_copy(s, n,
                   lambda r0, m: zeros_hbm.at[pl.ds(r0, m)],
                   lambda r0, m: deg_sh.at[pl.ds(r0, m)])
        for i in range(_CHUNK // 16):
            ones_v[pl.ds(i * 16, 16)] = jnp.ones((16,), jnp.float32)
        plsc.subcore_barrier()

        def body(j, carry):
            pltpu.sync_copy(ones_v, deg_sh.at[didx.at[j]], add=True)
            return carry

        lax.fori_loop(0, base, body, 0)

        @pl.when(wid < extra)
        def _():
            pltpu.sync_copy(
                dst2_hbm.at[pl.ds(base * _NW + wid, 1), :].at[0], xidx)
            pltpu.sync_copy(ones_v, deg_sh.at[xidx], add=True)

        plsc.subcore_barrier()
        _tile_copy(s, n,
                   lambda r0, m: deg_sh.at[pl.ds(r0, m)],
                   lambda r0, m: out_hbm.at[c, pl.ds(r0, m)])

    return degree


@functools.lru_cache(maxsize=None)
def _make_agg(n, e, f):
    nchunks = e // _CHUNK
    base = nchunks // _NW
    extra = nchunks - base * _NW
    npairs = base // 2
    odd_tail = base % 2

    @functools.partial(
        pl.kernel,
        out_type=jax.ShapeDtypeStruct((2, n, f), jnp.float32),
        mesh=_mesh(),
        scratch_types=[
            pltpu.VMEM((base, _CHUNK), jnp.int32),
            pltpu.VMEM((base, _CHUNK), jnp.int32),
            pltpu.VMEM((_CHUNK,), jnp.int32),
            pltpu.VMEM((_CHUNK,), jnp.int32),
            pltpu.VMEM((_CHUNK, f), jnp.float32),
            pltpu.VMEM((_CHUNK, f), jnp.float32),
            pltpu.VMEM_SHARED((n, f), jnp.float32),
            pltpu.SemaphoreType.DMA,
            pltpu.SemaphoreType.DMA,
        ],
        compiler_params=pltpu.CompilerParams(use_tc_tiling_on_sc=False),
    )
    def agg(hs_hbm, src2_hbm, dst2_hbm, out_hbm,
            sidx, didx, xsidx, xdidx, rows0, rows1, s_sh, sem0, sem1):
        c = lax.axis_index("c")
        s = lax.axis_index("s")
        wid = _worker_id()
        c0 = wid * base
        pltpu.sync_copy(src2_hbm.at[pl.ds(c0, base), :], sidx)
        pltpu.sync_copy(dst2_hbm.at[pl.ds(c0, base), :], didx)
        _tile_copy(s, n,
                   lambda r0, m: hs_hbm.at[pl.ds(r0, m), :],
                   lambda r0, m: s_sh.at[pl.ds(r0, m), :])
        plsc.subcore_barrier()

        @pl.when(wid < extra)
        def _():
            xc = base * _NW + wid
            pltpu.sync_copy(src2_hbm.at[pl.ds(xc, 1), :].at[0], xsidx)
            pltpu.sync_copy(dst2_hbm.at[pl.ds(xc, 1), :].at[0], xdidx)
            pltpu.async_copy(hs_hbm.at[xsidx], rows0, sem0).wait()
            pltpu.sync_copy(rows0, s_sh.at[xdidx], add=True)

        pltpu.async_copy(hs_hbm.at[sidx.at[0]], rows0, sem0)
        pltpu.async_copy(hs_hbm.at[sidx.at[1]], rows1, sem1)

        def body(i, carry):
            j0 = i * 2
            j1 = j0 + 1
            pltpu.make_async_copy(hs_hbm.at[sidx.at[j0]], rows0, sem0).wait()
            pltpu.sync_copy(rows0, s_sh.at[didx.at[j0]], add=True)

            @pl.when(j0 + 2 < base)
            def _():
                pltpu.async_copy(hs_hbm.at[sidx.at[j0 + 2]], rows0, sem0)

            pltpu.make_async_copy(hs_hbm.at[sidx.at[j1]], rows1, sem1).wait()
            pltpu.sync_copy(rows1, s_sh.at[didx.at[j1]], add=True)

            @pl.when(j1 + 2 < base)
            def _():
                pltpu.async_copy(hs_hbm.at[sidx.at[j1 + 2]], rows1, sem1)

            return carry

        lax.fori_loop(0, npairs, body, 0)
        if odd_tail:
            j = base - 1
            pltpu.make_async_copy(hs_hbm.at[sidx.at[j]], rows0, sem0).wait()
            pltpu.sync_copy(rows0, s_sh.at[didx.at[j]], add=True)

        plsc.subcore_barrier()
        _tile_copy(s, n,
                   lambda r0, m: s_sh.at[pl.ds(r0, m), :],
                   lambda r0, m: out_hbm.at[c, pl.ds(r0, m), :])

    return agg


def _prep1_body(x_ref, w1_ref, degp_ref, hs_ref, dinv_ref):
    deg = degp_ref[0] + degp_ref[1] + 1.0
    dinv = lax.rsqrt(deg)
    h = jnp.dot(x_ref[...], w1_ref[...], preferred_element_type=jnp.float32)
    hs_ref[...] = h * dinv
    dinv_ref[...] = dinv


def _mid_body(s1_ref, hs1_ref, dinv_ref, b1_ref, w2_ref, hs2_ref):
    dinv = dinv_ref[...]
    agg = s1_ref[0] + s1_ref[1] - hs1_ref[...]
    h = jnp.maximum(agg * dinv + b1_ref[...], 0.0)
    hs2_ref[...] = jnp.dot(h, w2_ref[...],
                           preferred_element_type=jnp.float32) * dinv


def _final_body(s2_ref, hs2_ref, dinv_ref, b2_ref, out_ref):
    o = (s2_ref[0] + s2_ref[1] - hs2_ref[...]) * dinv_ref[...] + b2_ref[...]
    m = jnp.max(o, axis=1, keepdims=True)
    z = o - m
    out_ref[...] = z - jnp.log(jnp.sum(jnp.exp(z), axis=1, keepdims=True))


def kernel(x, edge_index, W1, b1, W2, b2):
    n, d = x.shape
    h_dim = W1.shape[1]
    c_dim = W2.shape[1]
    e = edge_index.shape[1]
    src2 = edge_index[0].reshape(e // _CHUNK, _CHUNK)
    dst2 = edge_index[1].reshape(e // _CHUNK, _CHUNK)
    f32 = jnp.float32

    zeros_n = jnp.zeros((n,), f32)
    degp = _make_degree(n, e)(dst2, zeros_n)[:, :, None]

    bm = 1000
    grid = (n // bm,)
    hs1, dinv = pl.pallas_call(
        _prep1_body,
        grid=grid,
        in_specs=[
            pl.BlockSpec((bm, d), lambda i: (i, 0)),
            pl.BlockSpec((d, h_dim), lambda i: (0, 0)),
            pl.BlockSpec((2, bm, 1), lambda i: (0, i, 0)),
        ],
        out_specs=[
            pl.BlockSpec((bm, h_dim), lambda i: (i, 0)),
            pl.BlockSpec((bm, 1), lambda i: (i, 0)),
        ],
        out_shape=[
            jax.ShapeDtypeStruct((n, h_dim), f32),
            jax.ShapeDtypeStruct((n, 1), f32),
        ],
    )(x, W1, degp)

    s1 = _make_agg(n, e, h_dim)(hs1, src2, dst2)

    hs2 = pl.pallas_call(
        _mid_body,
        grid=grid,
        in_specs=[
            pl.BlockSpec((2, bm, h_dim), lambda i: (0, i, 0)),
            pl.BlockSpec((bm, h_dim), lambda i: (i, 0)),
            pl.BlockSpec((bm, 1), lambda i: (i, 0)),
            pl.BlockSpec((1, h_dim), lambda i: (0, 0)),
            pl.BlockSpec((h_dim, c_dim), lambda i: (0, 0)),
        ],
        out_specs=pl.BlockSpec((bm, c_dim), lambda i: (i, 0)),
        out_shape=jax.ShapeDtypeStruct((n, c_dim), f32),
    )(s1, hs1, dinv, b1.reshape(1, h_dim), W2)

    s2 = _make_agg(n, e, c_dim)(hs2, src2, dst2)

    out = pl.pallas_call(
        _final_body,
        grid=grid,
        in_specs=[
            pl.BlockSpec((2, bm, c_dim), lambda i: (0, i, 0)),
            pl.BlockSpec((bm, c_dim), lambda i: (i, 0)),
            pl.BlockSpec((bm, 1), lambda i: (i, 0)),
            pl.BlockSpec((1, c_dim), lambda i: (0, 0)),
        ],
        out_specs=pl.BlockSpec((bm, c_dim), lambda i: (i, 0)),
        out_shape=jax.ShapeDtypeStruct((n, c_dim), f32),
    )(s2, hs2, dinv, b2.reshape(1, c_dim))

    return out


# speedup vs baseline: 48.5858x; 1.1897x over previous
import functools

import jax
import jax.numpy as jnp
from jax import lax
from jax.experimental import pallas as pl
from jax.experimental.pallas import tpu as pltpu
from jax.experimental.pallas import tpu_sc as plsc

_CHUNK = 128
_NW = 32


def _worker_id():
    return lax.axis_index("c") * 16 + lax.axis_index("s")


def _row_split(n):
    rpt = ((n + 15) // 16 + 7) // 8 * 8
    last = n - 15 * rpt
    return rpt, last


def _tile_copy(s, n, src_fn, dst_fn):
    rpt, last = _row_split(n)
    row0 = pl.multiple_of(s * rpt, 8)

    @pl.when(s < 15)
    def _():
        pltpu.sync_copy(src_fn(row0, rpt), dst_fn(row0, rpt))

    @pl.when(s == 15)
    def _():
        pltpu.sync_copy(src_fn(15 * rpt, last), dst_fn(15 * rpt, last))


def _mesh():
    return plsc.VectorSubcoreMesh(core_axis_name="c", subcore_axis_name="s",
                                  num_cores=2, num_subcores=16)


@functools.lru_cache(maxsize=None)
def _make_degree(n, e):
    nchunks = e // _CHUNK
    base = nchunks // _NW
    extra = nchunks - base * _NW

    @functools.partial(
        pl.kernel,
        out_type=jax.ShapeDtypeStruct((2, n), jnp.float32),
        mesh=_mesh(),
        scratch_types=[
            pltpu.VMEM((base, _CHUNK), jnp.int32),
            pltpu.VMEM((_CHUNK,), jnp.int32),
            pltpu.VMEM((_CHUNK,), jnp.float32),
            pltpu.VMEM_SHARED((n,), jnp.float32),
        ],
        compiler_params=pltpu.CompilerParams(use_tc_tiling_on_sc=False),
    )
    def degree(edge3_hbm, zeros_hbm, out_hbm, didx, xidx, ones_v, deg_sh):
        c = lax.axis_index("c")
        s = lax.axis_index("s")
        wid = _worker_id()
        c0 = wid * base
        pltpu.sync_copy(edge3_hbm.at[1, pl.ds(c0, base), :], didx)
        _tile_copy(s, n,
                   lambda r0, m: zeros_hbm.at[pl.ds(r0, m)],
                   lambda r0, m: deg_sh.at[pl.ds(r0, m)])
        for i in range(_CHUNK // 16):
            ones_v[pl.ds(i * 16, 16)] = jnp.ones((16,), jnp.float32)
        plsc.subcore_barrier()

        def body(j, carry):
            pltpu.sync_copy(ones_v, deg_sh.at[didx.at[j]], add=True)
            return carry

        lax.fori_loop(0, base, body, 0)

        @pl.when(wid < extra)
        def _():
            pltpu.sync_copy(
                edge3_hbm.at[1, pl.ds(base * _NW + wid, 1), :].at[0], xidx)
            pltpu.sync_copy(ones_v, deg_sh.at[xidx], add=True)

        plsc.subcore_barrier()
        _tile_copy(s, n,
                   lambda r0, m: deg_sh.at[pl.ds(r0, m)],
                   lambda r0, m: out_hbm.at[c, pl.ds(r0, m)])

    return degree


@functools.lru_cache(maxsize=None)
def _make_agg(n, e, f):
    nchunks = e // _CHUNK
    base = nchunks // _NW
    extra = nchunks - base * _NW

    nbuf = 4
    ngroups = -(-base // nbuf)

    @functools.partial(
        pl.kernel,
        out_type=jax.ShapeDtypeStruct((2, n, f), jnp.float32),
        mesh=_mesh(),
        scratch_types=[
            pltpu.VMEM((base, _CHUNK), jnp.int32),
            pltpu.VMEM((base, _CHUNK), jnp.int32),
            pltpu.VMEM((_CHUNK,), jnp.int32),
            pltpu.VMEM((_CHUNK,), jnp.int32),
            [pltpu.VMEM((_CHUNK, f), jnp.float32)] * nbuf,
            [pltpu.SemaphoreType.DMA] * nbuf,
            [pltpu.SemaphoreType.DMA] * nbuf,
            pltpu.VMEM_SHARED((n, f), jnp.float32),
        ],
        compiler_params=pltpu.CompilerParams(use_tc_tiling_on_sc=False),
    )
    def agg(hs_hbm, edge3_hbm, out_hbm,
            sidx, didx, xsidx, xdidx, rows, gsem, ssem, s_sh):
        c = lax.axis_index("c")
        s = lax.axis_index("s")
        wid = _worker_id()
        c0 = wid * base
        pltpu.sync_copy(edge3_hbm.at[0, pl.ds(c0, base), :], sidx)
        pltpu.sync_copy(edge3_hbm.at[1, pl.ds(c0, base), :], didx)
        _tile_copy(s, n,
                   lambda r0, m: hs_hbm.at[pl.ds(r0, m), :],
                   lambda r0, m: s_sh.at[pl.ds(r0, m), :])
        plsc.subcore_barrier()

        @pl.when(wid < extra)
        def _():
            xc = base * _NW + wid
            pltpu.sync_copy(edge3_hbm.at[0, pl.ds(xc, 1), :].at[0], xsidx)
            pltpu.sync_copy(edge3_hbm.at[1, pl.ds(xc, 1), :].at[0], xdidx)
            pltpu.async_copy(hs_hbm.at[xsidx], rows[0], gsem[0]).wait()
            pltpu.sync_copy(rows[0], s_sh.at[xdidx], add=True)

        for k in range(nbuf):
            pltpu.async_copy(hs_hbm.at[sidx.at[k]], rows[k], gsem[k])

        def body(i, carry):
            j0 = i * nbuf
            for k in range(nbuf):
                jk = j0 + k

                @pl.when(jk < base)
                def _(k=k, jk=jk):
                    pltpu.make_async_copy(
                        hs_hbm.at[sidx.at[jk]], rows[k], gsem[k]).wait()
                    pltpu.async_copy(
                        rows[k], s_sh.at[didx.at[jk]], ssem[k], add=True)

            for k in range(nbuf):
                jk = j0 + k
                jn = jk + nbuf

                @pl.when(jk < base)
                def _(k=k, jk=jk):
                    pltpu.make_async_copy(
                        rows[k], s_sh.at[didx.at[jk]], ssem[k]).wait()

                @pl.when(jn < base)
                def _(k=k, jn=jn):
                    pltpu.async_copy(hs_hbm.at[sidx.at[jn]], rows[k], gsem[k])

            return carry

        lax.fori_loop(0, ngroups, body, 0)

        plsc.subcore_barrier()
        _tile_copy(s, n,
                   lambda r0, m: s_sh.at[pl.ds(r0, m), :],
                   lambda r0, m: out_hbm.at[c, pl.ds(r0, m), :])

    return agg


def _prep1_body(x_ref, w1_ref, degp_ref, hs_ref, dinv_ref):
    deg = degp_ref[0] + degp_ref[1] + 1.0
    dinv = lax.rsqrt(deg)
    h = jnp.dot(x_ref[...], w1_ref[...], preferred_element_type=jnp.float32)
    hs_ref[...] = h * dinv
    dinv_ref[...] = dinv


def _mid_body(s1_ref, hs1_ref, dinv_ref, b1_ref, w2_ref, hs2_ref):
    dinv = dinv_ref[...]
    agg = s1_ref[0] + s1_ref[1] - hs1_ref[...]
    h = jnp.maximum(agg * dinv + b1_ref[...], 0.0)
    hs2_ref[...] = jnp.dot(h, w2_ref[...],
                           preferred_element_type=jnp.float32) * dinv


def _final_body(s2_ref, hs2_ref, dinv_ref, b2_ref, out_ref):
    o = (s2_ref[0] + s2_ref[1] - hs2_ref[...]) * dinv_ref[...] + b2_ref[...]
    m = jnp.max(o, axis=1, keepdims=True)
    z = o - m
    out_ref[...] = z - jnp.log(jnp.sum(jnp.exp(z), axis=1, keepdims=True))


def kernel(x, edge_index, W1, b1, W2, b2):
    n, d = x.shape
    h_dim = W1.shape[1]
    c_dim = W2.shape[1]
    e = edge_index.shape[1]
    edge3 = edge_index.reshape(2, e // _CHUNK, _CHUNK)
    f32 = jnp.float32

    zeros_n = jnp.zeros((n,), f32)
    degp = _make_degree(n, e)(edge3, zeros_n)[:, :, None]

    bm = 1000
    grid = (n // bm,)
    hs1, dinv = pl.pallas_call(
        _prep1_body,
        grid=grid,
        in_specs=[
            pl.BlockSpec((bm, d), lambda i: (i, 0)),
            pl.BlockSpec((d, h_dim), lambda i: (0, 0)),
            pl.BlockSpec((2, bm, 1), lambda i: (0, i, 0)),
        ],
        out_specs=[
            pl.BlockSpec((bm, h_dim), lambda i: (i, 0)),
            pl.BlockSpec((bm, 1), lambda i: (i, 0)),
        ],
        out_shape=[
            jax.ShapeDtypeStruct((n, h_dim), f32),
            jax.ShapeDtypeStruct((n, 1), f32),
        ],
    )(x, W1, degp)

    s1 = _make_agg(n, e, h_dim)(hs1, edge3)

    hs2 = pl.pallas_call(
        _mid_body,
        grid=grid,
        in_specs=[
            pl.BlockSpec((2, bm, h_dim), lambda i: (0, i, 0)),
            pl.BlockSpec((bm, h_dim), lambda i: (i, 0)),
            pl.BlockSpec((bm, 1), lambda i: (i, 0)),
            pl.BlockSpec((1, h_dim), lambda i: (0, 0)),
            pl.BlockSpec((h_dim, c_dim), lambda i: (0, 0)),
        ],
        out_specs=pl.BlockSpec((bm, c_dim), lambda i: (i, 0)),
        out_shape=jax.ShapeDtypeStruct((n, c_dim), f32),
    )(s1, hs1, dinv, b1.reshape(1, h_dim), W2)

    s2 = _make_agg(n, e, c_dim)(hs2, edge3)

    out = pl.pallas_call(
        _final_body,
        grid=grid,
        in_specs=[
            pl.BlockSpec((2, bm, c_dim), lambda i: (0, i, 0)),
            pl.BlockSpec((bm, c_dim), lambda i: (i, 0)),
            pl.BlockSpec((bm, 1), lambda i: (i, 0)),
            pl.BlockSpec((1, c_dim), lambda i: (0, 0)),
        ],
        out_specs=pl.BlockSpec((bm, c_dim), lambda i: (i, 0)),
        out_shape=jax.ShapeDtypeStruct((n, c_dim), f32),
    )(s2, hs2, dinv, b2.reshape(1, c_dim))

    return out


# speedup vs baseline: 53.6739x; 1.1047x over previous
import functools

import jax
import jax.numpy as jnp
from jax import lax
from jax.experimental import pallas as pl
from jax.experimental.pallas import tpu as pltpu
from jax.experimental.pallas import tpu_sc as plsc

_CHUNK = 128
_NW = 32


def _worker_id():
    return lax.axis_index("c") * 16 + lax.axis_index("s")


def _row_split(n):
    rpt = ((n + 15) // 16 + 7) // 8 * 8
    last = n - 15 * rpt
    return rpt, last


def _tile_copy(s, n, src_fn, dst_fn):
    rpt, last = _row_split(n)
    row0 = pl.multiple_of(s * rpt, 8)

    @pl.when(s < 15)
    def _():
        pltpu.sync_copy(src_fn(row0, rpt), dst_fn(row0, rpt))

    @pl.when(s == 15)
    def _():
        pltpu.sync_copy(src_fn(15 * rpt, last), dst_fn(15 * rpt, last))


def _mesh():
    return plsc.VectorSubcoreMesh(core_axis_name="c", subcore_axis_name="s",
                                  num_cores=2, num_subcores=16)


@functools.lru_cache(maxsize=None)
def _make_degree(n, e):
    nchunks = e // _CHUNK
    base = nchunks // _NW
    extra = nchunks - base * _NW

    @functools.partial(
        pl.kernel,
        out_type=jax.ShapeDtypeStruct((2, n), jnp.float32),
        mesh=_mesh(),
        scratch_types=[
            pltpu.VMEM((base, _CHUNK), jnp.int32),
            pltpu.VMEM((_CHUNK,), jnp.int32),
            pltpu.VMEM((_CHUNK,), jnp.float32),
            pltpu.VMEM_SHARED((n,), jnp.float32),
        ],
        compiler_params=pltpu.CompilerParams(use_tc_tiling_on_sc=False),
    )
    def degree(edge3_hbm, zeros_hbm, out_hbm, didx, xidx, ones_v, deg_sh):
        c = lax.axis_index("c")
        s = lax.axis_index("s")
        wid = _worker_id()
        c0 = wid * base
        pltpu.sync_copy(edge3_hbm.at[1, pl.ds(c0, base), :], didx)
        _tile_copy(s, n,
                   lambda r0, m: zeros_hbm.at[pl.ds(r0, m)],
                   lambda r0, m: deg_sh.at[pl.ds(r0, m)])
        for i in range(_CHUNK // 16):
            ones_v[pl.ds(i * 16, 16)] = jnp.ones((16,), jnp.float32)
        plsc.subcore_barrier()

        def body(j, carry):
            pltpu.sync_copy(ones_v, deg_sh.at[didx.at[j]], add=True)
            return carry

        lax.fori_loop(0, base, body, 0)

        @pl.when(wid < extra)
        def _():
            pltpu.sync_copy(
                edge3_hbm.at[1, pl.ds(base * _NW + wid, 1), :].at[0], xidx)
            pltpu.sync_copy(ones_v, deg_sh.at[xidx], add=True)

        plsc.subcore_barrier()
        _tile_copy(s, n,
                   lambda r0, m: deg_sh.at[pl.ds(r0, m)],
                   lambda r0, m: out_hbm.at[c, pl.ds(r0, m)])

    return degree


@functools.lru_cache(maxsize=None)
def _make_agg(n, e, f):
    nchunks = e // _CHUNK
    base = nchunks // _NW
    extra = nchunks - base * _NW

    nbuf = 4
    ngroups = -(-base // nbuf)

    @functools.partial(
        pl.kernel,
        out_type=jax.ShapeDtypeStruct((2, n, f), jnp.float32),
        mesh=_mesh(),
        scratch_types=[
            pltpu.VMEM((base, _CHUNK), jnp.int32),
            pltpu.VMEM((base, _CHUNK), jnp.int32),
            pltpu.VMEM((_CHUNK,), jnp.int32),
            pltpu.VMEM((_CHUNK,), jnp.int32),
            [pltpu.VMEM((_CHUNK, f), jnp.float32)] * nbuf,
            [pltpu.SemaphoreType.DMA] * nbuf,
            [pltpu.SemaphoreType.DMA] * nbuf,
            pltpu.VMEM_SHARED((n, f), jnp.float32),
        ],
        compiler_params=pltpu.CompilerParams(use_tc_tiling_on_sc=False),
    )
    def agg(hs_hbm, edge3_hbm, out_hbm,
            sidx, didx, xsidx, xdidx, rows, gsem, ssem, s_sh):
        c = lax.axis_index("c")
        s = lax.axis_index("s")
        wid = _worker_id()
        c0 = wid * base
        pltpu.sync_copy(edge3_hbm.at[0, pl.ds(c0, base), :], sidx)
        pltpu.sync_copy(edge3_hbm.at[1, pl.ds(c0, base), :], didx)
        _tile_copy(s, n,
                   lambda r0, m: hs_hbm.at[pl.ds(r0, m), :],
                   lambda r0, m: s_sh.at[pl.ds(r0, m), :])
        plsc.subcore_barrier()

        @pl.when(wid < extra)
        def _():
            xc = base * _NW + wid
            pltpu.sync_copy(edge3_hbm.at[0, pl.ds(xc, 1), :].at[0], xsidx)
            pltpu.sync_copy(edge3_hbm.at[1, pl.ds(xc, 1), :].at[0], xdidx)
            pltpu.async_copy(hs_hbm.at[xsidx], rows[0], gsem[0]).wait()
            pltpu.sync_copy(rows[0], s_sh.at[xdidx], add=True)

        for k in range(nbuf):
            pltpu.async_copy(hs_hbm.at[sidx.at[k]], rows[k], gsem[k])

        def body(i, carry):
            j0 = i * nbuf
            for k in range(nbuf):
                jk = j0 + k

                @pl.when(jk < base)
                def _(k=k, jk=jk):
                    pltpu.make_async_copy(
                        hs_hbm.at[sidx.at[jk]], rows[k], gsem[k]).wait()
                    pltpu.async_copy(
                        rows[k], s_sh.at[didx.at[jk]], ssem[k], add=True)

            for k in range(nbuf):
                jk = j0 + k
                jn = jk + nbuf

                @pl.when(jk < base)
                def _(k=k, jk=jk):
                    pltpu.make_async_copy(
                        rows[k], s_sh.at[didx.at[jk]], ssem[k]).wait()

                @pl.when(jn < base)
                def _(k=k, jn=jn):
                    pltpu.async_copy(hs_hbm.at[sidx.at[jn]], rows[k], gsem[k])

            return carry

        lax.fori_loop(0, ngroups, body, 0)

        plsc.subcore_barrier()
        _tile_copy(s, n,
                   lambda r0, m: s_sh.at[pl.ds(r0, m), :],
                   lambda r0, m: out_hbm.at[c, pl.ds(r0, m), :])

    return agg


def _dinv_col(degp_blk):
    ones2 = jnp.ones((2, 1), jnp.float32)
    deg = lax.dot_general(degp_blk, ones2, (((0,), (0,)), ((), ())),
                          preferred_element_type=jnp.float32) + 1.0
    return lax.rsqrt(deg)


def _prep1_body(x_ref, w1_ref, degp_ref, hs_ref):
    dinv = _dinv_col(degp_ref[0])
    h = jnp.dot(x_ref[...], w1_ref[...], preferred_element_type=jnp.float32)
    hs_ref[...] = h * dinv


def _mid_body(s1_ref, hs1_ref, degp_ref, b1_ref, w2_ref, hs2_ref):
    dinv = _dinv_col(degp_ref[0])
    agg = s1_ref[0] + s1_ref[1] - hs1_ref[...]
    h = jnp.maximum(agg * dinv + b1_ref[...], 0.0)
    hs2_ref[...] = jnp.dot(h, w2_ref[...],
                           preferred_element_type=jnp.float32) * dinv


def _final_body(s2_ref, hs2_ref, degp_ref, b2_ref, out_ref):
    dinv = _dinv_col(degp_ref[0])
    o = (s2_ref[0] + s2_ref[1] - hs2_ref[...]) * dinv + b2_ref[...]
    m = jnp.max(o, axis=1, keepdims=True)
    z = o - m
    out_ref[...] = z - jnp.log(jnp.sum(jnp.exp(z), axis=1, keepdims=True))


def kernel(x, edge_index, W1, b1, W2, b2):
    n, d = x.shape
    h_dim = W1.shape[1]
    c_dim = W2.shape[1]
    e = edge_index.shape[1]
    edge3 = edge_index.reshape(2, e // _CHUNK, _CHUNK)
    f32 = jnp.float32

    zeros_n = jnp.zeros((n,), f32)
    bm = 2000
    grid = (n // bm,)
    degp = _make_degree(n, e)(edge3, zeros_n)
    degp = degp.reshape(2, n // bm, bm).swapaxes(0, 1)
    hs1 = pl.pallas_call(
        _prep1_body,
        grid=grid,
        in_specs=[
            pl.BlockSpec((bm, d), lambda i: (i, 0)),
            pl.BlockSpec((d, h_dim), lambda i: (0, 0)),
            pl.BlockSpec((1, 2, bm), lambda i: (i, 0, 0)),
        ],
        out_specs=pl.BlockSpec((bm, h_dim), lambda i: (i, 0)),
        out_shape=jax.ShapeDtypeStruct((n, h_dim), f32),
    )(x, W1, degp)

    s1 = _make_agg(n, e, h_dim)(hs1, edge3)

    hs2 = pl.pallas_call(
        _mid_body,
        grid=grid,
        in_specs=[
            pl.BlockSpec((2, bm, h_dim), lambda i: (0, i, 0)),
            pl.BlockSpec((bm, h_dim), lambda i: (i, 0)),
            pl.BlockSpec((1, 2, bm), lambda i: (i, 0, 0)),
            pl.BlockSpec((1, h_dim), lambda i: (0, 0)),
            pl.BlockSpec((h_dim, c_dim), lambda i: (0, 0)),
        ],
        out_specs=pl.BlockSpec((bm, c_dim), lambda i: (i, 0)),
        out_shape=jax.ShapeDtypeStruct((n, c_dim), f32),
    )(s1, hs1, degp, b1.reshape(1, h_dim), W2)

    s2 = _make_agg(n, e, c_dim)(hs2, edge3)

    out = pl.pallas_call(
        _final_body,
        grid=grid,
        in_specs=[
            pl.BlockSpec((2, bm, c_dim), lambda i: (0, i, 0)),
            pl.BlockSpec((bm, c_dim), lambda i: (i, 0)),
            pl.BlockSpec((1, 2, bm), lambda i: (i, 0, 0)),
            pl.BlockSpec((1, c_dim), lambda i: (0, 0)),
        ],
        out_specs=pl.BlockSpec((bm, c_dim), lambda i: (i, 0)),
        out_shape=jax.ShapeDtypeStruct((n, c_dim), f32),
    )(s2, hs2, degp, b2.reshape(1, c_dim))

    return out


# speedup vs baseline: 55.9169x; 1.0418x over previous
import functools

import jax
import jax.numpy as jnp
from jax import lax
from jax.experimental import pallas as pl
from jax.experimental.pallas import tpu as pltpu
from jax.experimental.pallas import tpu_sc as plsc

_CHUNK = 128
_NW = 32


def _worker_id():
    return lax.axis_index("c") * 16 + lax.axis_index("s")


def _row_split(n):
    rpt = ((n + 15) // 16 + 7) // 8 * 8
    last = n - 15 * rpt
    return rpt, last


def _tile_copy(s, n, src_fn, dst_fn):
    rpt, last = _row_split(n)
    row0 = pl.multiple_of(s * rpt, 8)

    @pl.when(s < 15)
    def _():
        pltpu.sync_copy(src_fn(row0, rpt), dst_fn(row0, rpt))

    @pl.when(s == 15)
    def _():
        pltpu.sync_copy(src_fn(15 * rpt, last), dst_fn(15 * rpt, last))


def _mesh():
    return plsc.VectorSubcoreMesh(core_axis_name="c", subcore_axis_name="s",
                                  num_cores=2, num_subcores=16)


@functools.lru_cache(maxsize=None)
def _make_degree(n, e):
    nchunks = e // _CHUNK
    base = nchunks // _NW
    extra = nchunks - base * _NW

    @functools.partial(
        pl.kernel,
        out_type=jax.ShapeDtypeStruct((2, n), jnp.float32),
        mesh=_mesh(),
        scratch_types=[
            pltpu.VMEM((base, _CHUNK), jnp.int32),
            pltpu.VMEM((_CHUNK,), jnp.int32),
            pltpu.VMEM((_CHUNK,), jnp.float32),
            pltpu.VMEM_SHARED((n,), jnp.float32),
        ],
        compiler_params=pltpu.CompilerParams(use_tc_tiling_on_sc=False),
    )
    def degree(edge3_hbm, zeros_hbm, out_hbm, didx, xidx, ones_v, deg_sh):
        c = lax.axis_index("c")
        s = lax.axis_index("s")
        wid = _worker_id()
        c0 = wid * base
        pltpu.sync_copy(edge3_hbm.at[1, pl.ds(c0, base), :], didx)
        _tile_copy(s, n,
                   lambda r0, m: zeros_hbm.at[pl.ds(r0, m)],
                   lambda r0, m: deg_sh.at[pl.ds(r0, m)])
        for i in range(_CHUNK // 16):
            ones_v[pl.ds(i * 16, 16)] = jnp.ones((16,), jnp.float32)
        plsc.subcore_barrier()

        def body(j, carry):
            pltpu.sync_copy(ones_v, deg_sh.at[didx.at[j]], add=True)
            return carry

        lax.fori_loop(0, base, body, 0)

        @pl.when(wid < extra)
        def _():
            pltpu.sync_copy(
                edge3_hbm.at[1, pl.ds(base * _NW + wid, 1), :].at[0], xidx)
            pltpu.sync_copy(ones_v, deg_sh.at[xidx], add=True)

        plsc.subcore_barrier()
        _tile_copy(s, n,
                   lambda r0, m: deg_sh.at[pl.ds(r0, m)],
                   lambda r0, m: out_hbm.at[c, pl.ds(r0, m)])

    return degree


@functools.lru_cache(maxsize=None)
def _make_agg(n, e, f):
    nchunks = e // _CHUNK
    base = nchunks // _NW
    extra = nchunks - base * _NW

    nbuf = 8
    ngroups = -(-base // nbuf)

    @functools.partial(
        pl.kernel,
        out_type=jax.ShapeDtypeStruct((2, n, f), jnp.float32),
        mesh=_mesh(),
        scratch_types=[
            pltpu.VMEM((base, _CHUNK), jnp.int32),
            pltpu.VMEM((base, _CHUNK), jnp.int32),
            pltpu.VMEM((_CHUNK,), jnp.int32),
            pltpu.VMEM((_CHUNK,), jnp.int32),
            [pltpu.VMEM((_CHUNK, f), jnp.float32)] * nbuf,
            [pltpu.SemaphoreType.DMA] * nbuf,
            [pltpu.SemaphoreType.DMA] * nbuf,
            pltpu.VMEM_SHARED((n, f), jnp.float32),
        ],
        compiler_params=pltpu.CompilerParams(use_tc_tiling_on_sc=False),
    )
    def agg(hs_hbm, edge3_hbm, out_hbm,
            sidx, didx, xsidx, xdidx, rows, gsem, ssem, s_sh):
        c = lax.axis_index("c")
        s = lax.axis_index("s")
        wid = _worker_id()
        c0 = wid * base
        pltpu.sync_copy(edge3_hbm.at[0, pl.ds(c0, base), :], sidx)
        pltpu.sync_copy(edge3_hbm.at[1, pl.ds(c0, base), :], didx)
        _tile_copy(s, n,
                   lambda r0, m: hs_hbm.at[pl.ds(r0, m), :],
                   lambda r0, m: s_sh.at[pl.ds(r0, m), :])
        plsc.subcore_barrier()

        @pl.when(wid < extra)
        def _():
            xc = base * _NW + wid
            pltpu.sync_copy(edge3_hbm.at[0, pl.ds(xc, 1), :].at[0], xsidx)
            pltpu.sync_copy(edge3_hbm.at[1, pl.ds(xc, 1), :].at[0], xdidx)
            pltpu.async_copy(hs_hbm.at[xsidx], rows[0], gsem[0]).wait()
            pltpu.sync_copy(rows[0], s_sh.at[xdidx], add=True)

        for k in range(nbuf):
            pltpu.async_copy(hs_hbm.at[sidx.at[k]], rows[k], gsem[k])

        def body(i, carry):
            j0 = i * nbuf
            for k in range(nbuf):
                jk = j0 + k

                @pl.when(jk < base)
                def _(k=k, jk=jk):
                    pltpu.make_async_copy(
                        hs_hbm.at[sidx.at[jk]], rows[k], gsem[k]).wait()
                    pltpu.async_copy(
                        rows[k], s_sh.at[didx.at[jk]], ssem[k], add=True)

            for k in range(nbuf):
                jk = j0 + k
                jn = jk + nbuf

                @pl.when(jk < base)
                def _(k=k, jk=jk):
                    pltpu.make_async_copy(
                        rows[k], s_sh.at[didx.at[jk]], ssem[k]).wait()

                @pl.when(jn < base)
                def _(k=k, jn=jn):
                    pltpu.async_copy(hs_hbm.at[sidx.at[jn]], rows[k], gsem[k])

            return carry

        lax.fori_loop(0, ngroups, body, 0)

        plsc.subcore_barrier()
        _tile_copy(s, n,
                   lambda r0, m: s_sh.at[pl.ds(r0, m), :],
                   lambda r0, m: out_hbm.at[c, pl.ds(r0, m), :])

    return agg


def _dinv_col(degp_blk):
    ones2 = jnp.ones((2, 1), jnp.float32)
    deg = lax.dot_general(degp_blk, ones2, (((0,), (0,)), ((), ())),
                          preferred_element_type=jnp.float32) + 1.0
    return lax.rsqrt(deg)


def _prep1_body(x_ref, w1_ref, degp_ref, hs_ref):
    dinv = _dinv_col(degp_ref[0])
    h = jnp.dot(x_ref[...], w1_ref[...], preferred_element_type=jnp.float32)
    hs_ref[...] = h * dinv


def _mid_body(s1_ref, hs1_ref, degp_ref, b1_ref, w2_ref, hs2_ref):
    dinv = _dinv_col(degp_ref[0])
    agg = s1_ref[0] + s1_ref[1] - hs1_ref[...]
    h = jnp.maximum(agg * dinv + b1_ref[...], 0.0)
    hs2_ref[...] = jnp.dot(h, w2_ref[...],
                           preferred_element_type=jnp.float32) * dinv


def _final_body(s2_ref, hs2_ref, degp_ref, b2_ref, out_ref):
    dinv = _dinv_col(degp_ref[0])
    o = (s2_ref[0] + s2_ref[1] - hs2_ref[...]) * dinv + b2_ref[...]
    m = jnp.max(o, axis=1, keepdims=True)
    z = o - m
    out_ref[...] = z - jnp.log(jnp.sum(jnp.exp(z), axis=1, keepdims=True))


def kernel(x, edge_index, W1, b1, W2, b2):
    n, d = x.shape
    h_dim = W1.shape[1]
    c_dim = W2.shape[1]
    e = edge_index.shape[1]
    edge3 = edge_index.reshape(2, e // _CHUNK, _CHUNK)
    f32 = jnp.float32

    zeros_n = jnp.zeros((n,), f32)
    bm = 2000
    grid = (n // bm,)
    degp = _make_degree(n, e)(edge3, zeros_n)
    degp = degp.reshape(2, n // bm, bm).swapaxes(0, 1)
    hs1 = pl.pallas_call(
        _prep1_body,
        grid=grid,
        in_specs=[
            pl.BlockSpec((bm, d), lambda i: (i, 0)),
            pl.BlockSpec((d, h_dim), lambda i: (0, 0)),
            pl.BlockSpec((1, 2, bm), lambda i: (i, 0, 0)),
        ],
        out_specs=pl.BlockSpec((bm, h_dim), lambda i: (i, 0)),
        out_shape=jax.ShapeDtypeStruct((n, h_dim), f32),
    )(x, W1, degp)

    s1 = _make_agg(n, e, h_dim)(hs1, edge3)

    hs2 = pl.pallas_call(
        _mid_body,
        grid=grid,
        in_specs=[
            pl.BlockSpec((2, bm, h_dim), lambda i: (0, i, 0)),
            pl.BlockSpec((bm, h_dim), lambda i: (i, 0)),
            pl.BlockSpec((1, 2, bm), lambda i: (i, 0, 0)),
            pl.BlockSpec((1, h_dim), lambda i: (0, 0)),
            pl.BlockSpec((h_dim, c_dim), lambda i: (0, 0)),
        ],
        out_specs=pl.BlockSpec((bm, c_dim), lambda i: (i, 0)),
        out_shape=jax.ShapeDtypeStruct((n, c_dim), f32),
    )(s1, hs1, degp, b1.reshape(1, h_dim), W2)

    s2 = _make_agg(n, e, c_dim)(hs2, edge3)

    out = pl.pallas_call(
        _final_body,
        grid=grid,
        in_specs=[
            pl.BlockSpec((2, bm, c_dim), lambda i: (0, i, 0)),
            pl.BlockSpec((bm, c_dim), lambda i: (i, 0)),
            pl.BlockSpec((1, 2, bm), lambda i: (i, 0, 0)),
            pl.BlockSpec((1, c_dim), lambda i: (0, 0)),
        ],
        out_specs=pl.BlockSpec((bm, c_dim), lambda i: (i, 0)),
        out_shape=jax.ShapeDtypeStruct((n, c_dim), f32),
    )(s2, hs2, degp, b2.reshape(1, c_dim))

    return out


# speedup vs baseline: 56.2874x; 1.0066x over previous
import functools

import jax
import jax.numpy as jnp
from jax import lax
from jax.experimental import pallas as pl
from jax.experimental.pallas import tpu as pltpu
from jax.experimental.pallas import tpu_sc as plsc

_CHUNK = 128
_NW = 32


def _worker_id():
    return lax.axis_index("c") * 16 + lax.axis_index("s")


def _row_split(n):
    rpt = ((n + 15) // 16 + 7) // 8 * 8
    last = n - 15 * rpt
    return rpt, last


def _tile_copy(s, n, src_fn, dst_fn):
    rpt, last = _row_split(n)
    row0 = pl.multiple_of(s * rpt, 8)

    @pl.when(s < 15)
    def _():
        pltpu.sync_copy(src_fn(row0, rpt), dst_fn(row0, rpt))

    @pl.when(s == 15)
    def _():
        pltpu.sync_copy(src_fn(15 * rpt, last), dst_fn(15 * rpt, last))


def _mesh():
    return plsc.VectorSubcoreMesh(core_axis_name="c", subcore_axis_name="s",
                                  num_cores=2, num_subcores=16)


@functools.lru_cache(maxsize=None)
def _make_degree(n, e):
    nchunks = e // _CHUNK
    base = nchunks // _NW
    extra = nchunks - base * _NW

    @functools.partial(
        pl.kernel,
        out_type=jax.ShapeDtypeStruct((2, n), jnp.float32),
        mesh=_mesh(),
        scratch_types=[
            pltpu.VMEM((base, _CHUNK), jnp.int32),
            pltpu.VMEM((_CHUNK,), jnp.int32),
            pltpu.VMEM((_CHUNK,), jnp.float32),
            pltpu.VMEM_SHARED((n,), jnp.float32),
        ],
        compiler_params=pltpu.CompilerParams(use_tc_tiling_on_sc=False),
    )
    def degree(edge3_hbm, zeros_hbm, out_hbm, didx, xidx, ones_v, deg_sh):
        c = lax.axis_index("c")
        s = lax.axis_index("s")
        wid = _worker_id()
        c0 = wid * base
        pltpu.sync_copy(edge3_hbm.at[1, pl.ds(c0, base), :], didx)
        _tile_copy(s, n,
                   lambda r0, m: zeros_hbm.at[pl.ds(r0, m)],
                   lambda r0, m: deg_sh.at[pl.ds(r0, m)])
        for i in range(_CHUNK // 16):
            ones_v[pl.ds(i * 16, 16)] = jnp.ones((16,), jnp.float32)
        plsc.subcore_barrier()

        def body(j, carry):
            pltpu.sync_copy(ones_v, deg_sh.at[didx.at[j]], add=True)
            return carry

        lax.fori_loop(0, base, body, 0)

        @pl.when(wid < extra)
        def _():
            pltpu.sync_copy(
                edge3_hbm.at[1, pl.ds(base * _NW + wid, 1), :].at[0], xidx)
            pltpu.sync_copy(ones_v, deg_sh.at[xidx], add=True)

        plsc.subcore_barrier()
        _tile_copy(s, n,
                   lambda r0, m: deg_sh.at[pl.ds(r0, m)],
                   lambda r0, m: out_hbm.at[c, pl.ds(r0, m)])

    return degree


@functools.lru_cache(maxsize=None)
def _make_agg(n, e, f):
    nchunks = e // _CHUNK
    base = nchunks // _NW
    extra = nchunks - base * _NW

    nbuf = 12
    ngroups = -(-base // nbuf)

    @functools.partial(
        pl.kernel,
        out_type=jax.ShapeDtypeStruct((2, n, f), jnp.float32),
        mesh=_mesh(),
        scratch_types=[
            pltpu.VMEM((base, _CHUNK), jnp.int32),
            pltpu.VMEM((base, _CHUNK), jnp.int32),
            pltpu.VMEM((_CHUNK,), jnp.int32),
            pltpu.VMEM((_CHUNK,), jnp.int32),
            [pltpu.VMEM((_CHUNK, f), jnp.float32)] * nbuf,
            [pltpu.SemaphoreType.DMA] * nbuf,
            [pltpu.SemaphoreType.DMA] * nbuf,
            pltpu.VMEM_SHARED((n, f), jnp.float32),
        ],
        compiler_params=pltpu.CompilerParams(use_tc_tiling_on_sc=False),
    )
    def agg(hs_hbm, edge3_hbm, out_hbm,
            sidx, didx, xsidx, xdidx, rows, gsem, ssem, s_sh):
        c = lax.axis_index("c")
        s = lax.axis_index("s")
        wid = _worker_id()
        c0 = wid * base
        pltpu.sync_copy(edge3_hbm.at[0, pl.ds(c0, base), :], sidx)
        pltpu.sync_copy(edge3_hbm.at[1, pl.ds(c0, base), :], didx)
        _tile_copy(s, n,
                   lambda r0, m: hs_hbm.at[pl.ds(r0, m), :],
                   lambda r0, m: s_sh.at[pl.ds(r0, m), :])
        plsc.subcore_barrier()

        @pl.when(wid < extra)
        def _():
            xc = base * _NW + wid
            pltpu.sync_copy(edge3_hbm.at[0, pl.ds(xc, 1), :].at[0], xsidx)
            pltpu.sync_copy(edge3_hbm.at[1, pl.ds(xc, 1), :].at[0], xdidx)
            pltpu.async_copy(hs_hbm.at[xsidx], rows[0], gsem[0]).wait()
            pltpu.sync_copy(rows[0], s_sh.at[xdidx], add=True)

        for k in range(nbuf):
            pltpu.async_copy(hs_hbm.at[sidx.at[k]], rows[k], gsem[k])

        def body(i, carry):
            j0 = i * nbuf
            for k in range(nbuf):
                jk = j0 + k

                @pl.when(jk < base)
                def _(k=k, jk=jk):
                    pltpu.make_async_copy(
                        hs_hbm.at[sidx.at[jk]], rows[k], gsem[k]).wait()
                    pltpu.async_copy(
                        rows[k], s_sh.at[didx.at[jk]], ssem[k], add=True)

            for k in range(nbuf):
                jk = j0 + k
                jn = jk + nbuf

                @pl.when(jk < base)
                def _(k=k, jk=jk):
                    pltpu.make_async_copy(
                        rows[k], s_sh.at[didx.at[jk]], ssem[k]).wait()

                @pl.when(jn < base)
                def _(k=k, jn=jn):
                    pltpu.async_copy(hs_hbm.at[sidx.at[jn]], rows[k], gsem[k])

            return carry

        lax.fori_loop(0, ngroups, body, 0)

        plsc.subcore_barrier()
        _tile_copy(s, n,
                   lambda r0, m: s_sh.at[pl.ds(r0, m), :],
                   lambda r0, m: out_hbm.at[c, pl.ds(r0, m), :])

    return agg


def _dinv_col(degp_blk):
    ones2 = jnp.ones((2, 1), jnp.float32)
    deg = lax.dot_general(degp_blk, ones2, (((0,), (0,)), ((), ())),
                          preferred_element_type=jnp.float32) + 1.0
    return lax.rsqrt(deg)


def _prep1_body(x_ref, w1_ref, degp_ref, hs_ref):
    dinv = _dinv_col(degp_ref[0])
    h = jnp.dot(x_ref[...], w1_ref[...], preferred_element_type=jnp.float32)
    hs_ref[...] = h * dinv


def _mid_body(s1_ref, hs1_ref, degp_ref, b1_ref, w2_ref, hs2_ref):
    dinv = _dinv_col(degp_ref[0])
    agg = s1_ref[0] + s1_ref[1] - hs1_ref[...]
    h = jnp.maximum(agg * dinv + b1_ref[...], 0.0)
    hs2_ref[...] = jnp.dot(h, w2_ref[...],
                           preferred_element_type=jnp.float32) * dinv


def _final_body(s2_ref, hs2_ref, degp_ref, b2_ref, out_ref):
    dinv = _dinv_col(degp_ref[0])
    o = (s2_ref[0] + s2_ref[1] - hs2_ref[...]) * dinv + b2_ref[...]
    m = jnp.max(o, axis=1, keepdims=True)
    z = o - m
    out_ref[...] = z - jnp.log(jnp.sum(jnp.exp(z), axis=1, keepdims=True))


def kernel(x, edge_index, W1, b1, W2, b2):
    n, d = x.shape
    h_dim = W1.shape[1]
    c_dim = W2.shape[1]
    e = edge_index.shape[1]
    edge3 = edge_index.reshape(2, e // _CHUNK, _CHUNK)
    f32 = jnp.float32

    zeros_n = jnp.zeros((n,), f32)
    bm = 2000
    grid = (n // bm,)
    degp = _make_degree(n, e)(edge3, zeros_n)
    degp = degp.reshape(2, n // bm, bm).swapaxes(0, 1)
    hs1 = pl.pallas_call(
        _prep1_body,
        grid=grid,
        in_specs=[
            pl.BlockSpec((bm, d), lambda i: (i, 0)),
            pl.BlockSpec((d, h_dim), lambda i: (0, 0)),
            pl.BlockSpec((1, 2, bm), lambda i: (i, 0, 0)),
        ],
        out_specs=pl.BlockSpec((bm, h_dim), lambda i: (i, 0)),
        out_shape=jax.ShapeDtypeStruct((n, h_dim), f32),
    )(x, W1, degp)

    s1 = _make_agg(n, e, h_dim)(hs1, edge3)

    hs2 = pl.pallas_call(
        _mid_body,
        grid=grid,
        in_specs=[
            pl.BlockSpec((2, bm, h_dim), lambda i: (0, i, 0)),
            pl.BlockSpec((bm, h_dim), lambda i: (i, 0)),
            pl.BlockSpec((1, 2, bm), lambda i: (i, 0, 0)),
            pl.BlockSpec((1, h_dim), lambda i: (0, 0)),
            pl.BlockSpec((h_dim, c_dim), lambda i: (0, 0)),
        ],
        out_specs=pl.BlockSpec((bm, c_dim), lambda i: (i, 0)),
        out_shape=jax.ShapeDtypeStruct((n, c_dim), f32),
    )(s1, hs1, degp, b1.reshape(1, h_dim), W2)

    s2 = _make_agg(n, e, c_dim)(hs2, edge3)

    out = pl.pallas_call(
        _final_body,
        grid=grid,
        in_specs=[
            pl.BlockSpec((2, bm, c_dim), lambda i: (0, i, 0)),
            pl.BlockSpec((bm, c_dim), lambda i: (i, 0)),
            pl.BlockSpec((1, 2, bm), lambda i: (i, 0, 0)),
            pl.BlockSpec((1, c_dim), lambda i: (0, 0)),
        ],
        out_specs=pl.BlockSpec((bm, c_dim), lambda i: (i, 0)),
        out_shape=jax.ShapeDtypeStruct((n, c_dim), f32),
    )(s2, hs2, degp, b2.reshape(1, c_dim))

    return out
